# initial kernel scaffold (unmeasured)
import jax
import jax.numpy as jnp
from jax import lax
from jax.experimental import pallas as pl
from jax.experimental.pallas import tpu as pltpu

N_DEV = 32
BLK = 64


def kernel(x, w_mat):
    k_total, m_per = x.shape
    n = w_mat.shape[1]
    assert m_per == BLK and k_total == N_DEV * BLK

    def body(x_ref, w_ref, out_ref, gathered_ref, send_sems, recv_sems):
        my_pos = lax.axis_index("i")

        gathered_ref[my_pos] = x_ref[pl.ds(my_pos * BLK, BLK), :]

        rdmas = []
        for k in range(1, N_DEV):
            tgt = lax.rem(my_pos + k, N_DEV)
            rdma = pltpu.make_async_remote_copy(
                src_ref=x_ref.at[pl.ds(tgt * BLK, BLK), :],
                dst_ref=gathered_ref.at[my_pos],
                send_sem=send_sems.at[k],
                recv_sem=recv_sems.at[my_pos],
                device_id=(tgt,),
                device_id_type=pl.DeviceIdType.MESH,
            )
            rdma.start()
            rdmas.append(rdma)

        for k in range(1, N_DEV):
            src = lax.rem(my_pos + N_DEV - k, N_DEV)
            recv = pltpu.make_async_remote_copy(
                src_ref=x_ref.at[pl.ds(0, BLK), :],
                dst_ref=gathered_ref.at[src],
                send_sem=send_sems.at[0],
                recv_sem=recv_sems.at[src],
                device_id=(src,),
                device_id_type=pl.DeviceIdType.MESH,
            )
            recv.wait_recv()

        for rdma in rdmas:
            rdma.wait_send()

        x_rows = jnp.swapaxes(gathered_ref[...], 0, 1).reshape(BLK, k_total)
        y = jnp.dot(x_rows, w_ref[...], preferred_element_type=jnp.float32)
        out_ref[...] = y * jax.nn.sigmoid(y)

    return pl.pallas_call(
        body,
        out_shape=jax.ShapeDtypeStruct((BLK, n), jnp.float32),
        in_specs=[
            pl.BlockSpec(memory_space=pltpu.VMEM),
            pl.BlockSpec(memory_space=pltpu.VMEM),
        ],
        out_specs=pl.BlockSpec(memory_space=pltpu.VMEM),
        scratch_shapes=[
            pltpu.VMEM((N_DEV, BLK, BLK), jnp.float32),
            pltpu.SemaphoreType.DMA((N_DEV,)),
            pltpu.SemaphoreType.DMA((N_DEV,)),
        ],
        compiler_params=pltpu.CompilerParams(collective_id=0),
    )(x, w_mat)


# baseline (device time: 34972 ns/iter reference)
import jax
import jax.numpy as jnp
from jax import lax
from jax.experimental import pallas as pl
from jax.experimental.pallas import tpu as pltpu

N_DEV = 32
BLK = 64


def kernel(x, w_mat):
    k_total, m_per = x.shape
    n = w_mat.shape[1]
    assert m_per == BLK and k_total == N_DEV * BLK

    def body(x_ref, w_ref, out_ref, gathered_ref, send_sems, recv_sems):
        my_pos = lax.axis_index("i")

        gathered_ref[my_pos] = x_ref[pl.ds(my_pos * BLK, BLK), :]

        rdmas = []
        for k in range(1, N_DEV):
            tgt = lax.rem(my_pos + k, N_DEV)
            rdma = pltpu.make_async_remote_copy(
                src_ref=x_ref.at[pl.ds(tgt * BLK, BLK), :],
                dst_ref=gathered_ref.at[my_pos],
                send_sem=send_sems.at[k],
                recv_sem=recv_sems.at[my_pos],
                device_id=(tgt,),
                device_id_type=pl.DeviceIdType.MESH,
            )
            rdma.start()
            rdmas.append(rdma)

        for k in range(1, N_DEV):
            src = lax.rem(my_pos + N_DEV - k, N_DEV)
            recv = pltpu.make_async_remote_copy(
                src_ref=x_ref.at[pl.ds(0, BLK), :],
                dst_ref=gathered_ref.at[src],
                send_sem=send_sems.at[0],
                recv_sem=recv_sems.at[src],
                device_id=(src,),
                device_id_type=pl.DeviceIdType.MESH,
            )
            recv.wait_recv()

        for rdma in rdmas:
            rdma.wait_send()

        x_rows = jnp.swapaxes(gathered_ref[...], 0, 1).reshape(BLK, k_total)
        y = jnp.dot(x_rows, w_ref[...], preferred_element_type=jnp.float32)
        out_ref[...] = y * jax.nn.sigmoid(y)

    return pl.pallas_call(
        body,
        out_shape=jax.ShapeDtypeStruct((BLK, n), jnp.float32),
        in_specs=[
            pl.BlockSpec(memory_space=pltpu.VMEM),
            pl.BlockSpec(memory_space=pltpu.VMEM),
        ],
        out_specs=pl.BlockSpec(memory_space=pltpu.VMEM),
        scratch_shapes=[
            pltpu.VMEM((N_DEV, BLK, BLK), jnp.float32),
            pltpu.SemaphoreType.DMA((N_DEV,)),
            pltpu.SemaphoreType.DMA((N_DEV,)),
        ],
    )(x, w_mat)
